# baseline (speedup 1.0000x reference)
import functools

import jax
import jax.numpy as jnp
from jax import lax
from jax.experimental import pallas as pl
from jax.experimental.pallas import tpu as pltpu
from jax.experimental.pallas import tpu_sc as plsc

N = 10000
E = 320000
D = 128

NC = 2
NS = 16
NW = NC * NS
EPW = E // NW
CHUNK = 80
NCHUNK = EPW // CHUNK
GSZ = 25
NGRP = NCHUNK // GSZ
NBUF = 4
NACC = N
RPT = 624
TAIL = N - NS * RPT

_sc_mesh = plsc.VectorSubcoreMesh(core_axis_name="c", subcore_axis_name="s")


@functools.partial(
    pl.kernel,
    out_type=jax.ShapeDtypeStruct((NC * N, D), jnp.float32),
    mesh=_sc_mesh,
    scratch_types=[
        pltpu.VMEM_SHARED((NACC, D), jnp.float32),
        pltpu.VMEM((GSZ * CHUNK,), jnp.int32),
        pltpu.VMEM((GSZ, CHUNK), jnp.int32),
        pltpu.VMEM((NBUF, CHUNK, D), jnp.float32),
        pltpu.SemaphoreType.DMA,
        pltpu.SemaphoreType.DMA,
        pltpu.SemaphoreType.DMA,
        pltpu.SemaphoreType.DMA,
    ],
)
def _sc_aggregate(x_hbm, src_hbm, dst_hbm, out_hbm, acc, srcv, dstv, rows,
                  sem0, sem1, sem2, sem3):
    c = lax.axis_index("c")
    s = lax.axis_index("s")
    wid = c * NS + s

    pltpu.sync_copy(x_hbm.at[pl.ds(s * RPT, RPT)], acc.at[pl.ds(s * RPT, RPT)])

    @pl.when(s == NS - 1)
    def _seed_tail():
        pltpu.sync_copy(x_hbm.at[pl.ds(NS * RPT, TAIL)],
                        acc.at[pl.ds(NS * RPT, TAIL)])

    plsc.subcore_barrier()

    sems = (sem0, sem1, sem2, sem3)

    def group(g, carry):
        pltpu.sync_copy(src_hbm.at[wid, g], srcv)
        pltpu.sync_copy(dst_hbm.at[wid, g], dstv)
        for b in range(NBUF):
            pltpu.async_copy(x_hbm.at[srcv.at[pl.ds(b * CHUNK, CHUNK)]],
                             rows.at[b], sems[b])

        def chunk(j, c2):
            for b in range(NBUF):
                @pl.when(lax.rem(j, NBUF) == b)
                def _():
                    pltpu.make_async_copy(
                        x_hbm.at[srcv.at[pl.ds(j * CHUNK, CHUNK)]],
                        rows.at[b], sems[b]).wait()
                    pltpu.sync_copy(rows.at[b], acc.at[dstv.at[j]], add=True)

                    @pl.when(j + NBUF < GSZ)
                    def _():
                        pltpu.async_copy(
                            x_hbm.at[srcv.at[pl.ds((j + NBUF) * CHUNK, CHUNK)]],
                            rows.at[b], sems[b])
            return c2

        lax.fori_loop(0, GSZ, chunk, 0)
        return carry

    lax.fori_loop(0, NGRP, group, 0)
    plsc.subcore_barrier()
    pltpu.sync_copy(acc.at[pl.ds(s * RPT, RPT)],
                    out_hbm.at[pl.ds(c * N + s * RPT, RPT)])

    @pl.when(s == NS - 1)
    def _out_tail():
        pltpu.sync_copy(acc.at[pl.ds(NS * RPT, TAIL)],
                        out_hbm.at[pl.ds(c * N + NS * RPT, TAIL)])


ROWS_BLK = 2000


def _mlp_body(final, p0_ref, p1_ref, x_ref, wa_ref, ba_ref, wb_ref, bb_ref, o_ref):
    h = p0_ref[...] + p1_ref[...] - x_ref[...]
    t = jnp.dot(h, wa_ref[...], preferred_element_type=jnp.float32) + ba_ref[...]
    t = jnp.maximum(t, 0.0)
    o = jnp.dot(t, wb_ref[...], preferred_element_type=jnp.float32) + bb_ref[...]
    if final == "relu":
        o_ref[...] = jnp.maximum(o, 0.0)
    else:
        m = jnp.max(o, axis=1, keepdims=True)
        e = jnp.exp(o - m)
        o_ref[...] = o - (jnp.log(jnp.sum(e, axis=1, keepdims=True)) + m)


def _mlp(final, p, x, wa, ba, wb, bb):
    grid = N // ROWS_BLK
    return pl.pallas_call(
        functools.partial(_mlp_body, final),
        grid=(grid,),
        in_specs=[
            pl.BlockSpec((ROWS_BLK, D), lambda i: (i, 0)),
            pl.BlockSpec((ROWS_BLK, D), lambda i, g=grid: (i + g, 0)),
            pl.BlockSpec((ROWS_BLK, D), lambda i: (i, 0)),
            pl.BlockSpec((D, D), lambda i: (0, 0)),
            pl.BlockSpec((1, D), lambda i: (0, 0)),
            pl.BlockSpec((D, D), lambda i: (0, 0)),
            pl.BlockSpec((1, D), lambda i: (0, 0)),
        ],
        out_specs=pl.BlockSpec((ROWS_BLK, D), lambda i: (i, 0)),
        out_shape=jax.ShapeDtypeStruct((N, D), jnp.float32),
    )(p, p, x, wa, ba, wb, bb)


def kernel(x, edge_index, W1a, b1a, W1b, b1b, W2a, b2a, W2b, b2b):
    src = edge_index[0].astype(jnp.int32).reshape(NW, NGRP, GSZ * CHUNK)
    dst = edge_index[1].astype(jnp.int32).reshape(NW, NGRP, GSZ, CHUNK)
    b1a2, b1b2 = b1a.reshape(1, D), b1b.reshape(1, D)
    b2a2, b2b2 = b2a.reshape(1, D), b2b.reshape(1, D)

    p = _sc_aggregate(x, src, dst)
    h1 = _mlp("relu", p, x, W1a, b1a2, W1b, b1b2)
    p2 = _sc_aggregate(h1, src, dst)
    return _mlp("logsoftmax", p2, h1, W2a, b2a2, W2b, b2b2)


# speedup vs baseline: 1.1212x; 1.1212x over previous
import functools

import jax
import jax.numpy as jnp
from jax import lax
from jax.experimental import pallas as pl
from jax.experimental.pallas import tpu as pltpu
from jax.experimental.pallas import tpu_sc as plsc

N = 10000
E = 320000
D = 128

NC = 2
NS = 16
NW = NC * NS
EPW = E // NW
CHUNK = 80
NCHUNK = EPW // CHUNK
GSZ = 25
NGRP = NCHUNK // GSZ
NBUF = 3
NACC = N
RPT = 624
TAIL = N - NS * RPT

_sc_mesh = plsc.VectorSubcoreMesh(core_axis_name="c", subcore_axis_name="s")


@functools.partial(
    pl.kernel,
    out_type=jax.ShapeDtypeStruct((NC * N, D), jnp.float32),
    mesh=_sc_mesh,
    scratch_types=[
        pltpu.VMEM_SHARED((NACC, D), jnp.float32),
        pltpu.VMEM((2, GSZ, CHUNK), jnp.int32),
        pltpu.VMEM((2, GSZ, CHUNK), jnp.int32),
        pltpu.VMEM((NBUF, CHUNK, D), jnp.float32),
        pltpu.SemaphoreType.DMA,
        pltpu.SemaphoreType.DMA,
        pltpu.SemaphoreType.DMA,
        pltpu.SemaphoreType.DMA,
        pltpu.SemaphoreType.DMA,
        pltpu.SemaphoreType.DMA,
    ],
)
def _sc_aggregate(x_hbm, src_hbm, dst_hbm, out_hbm, acc, srcv, dstv, rows,
                  ssem, isem0, isem1, gsem0, gsem1, gsem2):
    c = lax.axis_index("c")
    s = lax.axis_index("s")
    wid = c * NS + s
    isems = (isem0, isem1)
    gsems = (gsem0, gsem1, gsem2)

    pltpu.async_copy(x_hbm.at[pl.ds(s * RPT, RPT)],
                     acc.at[pl.ds(s * RPT, RPT)], ssem)

    @pl.when(s == NS - 1)
    def _seed_tail():
        pltpu.async_copy(x_hbm.at[pl.ds(NS * RPT, TAIL)],
                         acc.at[pl.ds(NS * RPT, TAIL)], ssem)

    pltpu.async_copy(src_hbm.at[wid, 0], srcv.at[0], isem0)
    pltpu.async_copy(dst_hbm.at[wid, 0], dstv.at[0], isem0)
    pltpu.make_async_copy(src_hbm.at[wid, 0], srcv.at[0], isem0).wait()
    pltpu.make_async_copy(dst_hbm.at[wid, 0], dstv.at[0], isem0).wait()

    for b in range(NBUF):
        pltpu.async_copy(x_hbm.at[srcv.at[0, b]], rows.at[b], gsems[b])

    pltpu.make_async_copy(x_hbm.at[pl.ds(s * RPT, RPT)],
                          acc.at[pl.ds(s * RPT, RPT)], ssem).wait()

    @pl.when(s == NS - 1)
    def _seed_tail_wait():
        pltpu.make_async_copy(x_hbm.at[pl.ds(NS * RPT, TAIL)],
                              acc.at[pl.ds(NS * RPT, TAIL)], ssem).wait()

    plsc.subcore_barrier()

    def chunk(i, carry):
        g = lax.div(i, GSZ)
        j = lax.rem(i, GSZ)
        for gb in range(2):
            @pl.when(lax.rem(g, 2) == gb)
            def _():
                ob = 1 - gb

                @pl.when((j == 0) & (g + 1 < NGRP))
                def _():
                    pltpu.async_copy(src_hbm.at[wid, g + 1], srcv.at[ob],
                                     isems[ob])
                    pltpu.async_copy(dst_hbm.at[wid, g + 1], dstv.at[ob],
                                     isems[ob])

                for b in range(NBUF):
                    @pl.when(lax.rem(i, NBUF) == b)
                    def _():
                        pltpu.make_async_copy(
                            x_hbm.at[srcv.at[gb, j]],
                            rows.at[b], gsems[b]).wait()
                        pltpu.sync_copy(rows.at[b], acc.at[dstv.at[gb, j]],
                                        add=True)

                        @pl.when((i + NBUF < NCHUNK) & (j + NBUF < GSZ))
                        def _():
                            pltpu.async_copy(
                                x_hbm.at[srcv.at[gb, j + NBUF]],
                                rows.at[b], gsems[b])

                        @pl.when((i + NBUF < NCHUNK) & (j + NBUF >= GSZ))
                        def _():
                            @pl.when(j + NBUF == GSZ)
                            def _():
                                pltpu.make_async_copy(
                                    src_hbm.at[wid, g + 1], srcv.at[ob],
                                    isems[ob]).wait()
                                pltpu.make_async_copy(
                                    dst_hbm.at[wid, g + 1], dstv.at[ob],
                                    isems[ob]).wait()
                            pltpu.async_copy(
                                x_hbm.at[srcv.at[ob, j + NBUF - GSZ]],
                                rows.at[b], gsems[b])
        return carry

    lax.fori_loop(0, NCHUNK, chunk, 0)
    plsc.subcore_barrier()
    pltpu.sync_copy(acc.at[pl.ds(s * RPT, RPT)],
                    out_hbm.at[pl.ds(c * N + s * RPT, RPT)])

    @pl.when(s == NS - 1)
    def _out_tail():
        pltpu.sync_copy(acc.at[pl.ds(NS * RPT, TAIL)],
                        out_hbm.at[pl.ds(c * N + NS * RPT, TAIL)])


ROWS_BLK = 2000


def _mlp_body(final, p0_ref, p1_ref, x_ref, wa_ref, ba_ref, wb_ref, bb_ref, o_ref):
    h = p0_ref[...] + p1_ref[...] - x_ref[...]
    t = jnp.dot(h, wa_ref[...], preferred_element_type=jnp.float32) + ba_ref[...]
    t = jnp.maximum(t, 0.0)
    o = jnp.dot(t, wb_ref[...], preferred_element_type=jnp.float32) + bb_ref[...]
    if final == "relu":
        o_ref[...] = jnp.maximum(o, 0.0)
    else:
        m = jnp.max(o, axis=1, keepdims=True)
        e = jnp.exp(o - m)
        o_ref[...] = o - (jnp.log(jnp.sum(e, axis=1, keepdims=True)) + m)


def _mlp(final, p, x, wa, ba, wb, bb):
    grid = N // ROWS_BLK
    return pl.pallas_call(
        functools.partial(_mlp_body, final),
        grid=(grid,),
        in_specs=[
            pl.BlockSpec((ROWS_BLK, D), lambda i: (i, 0)),
            pl.BlockSpec((ROWS_BLK, D), lambda i, g=grid: (i + g, 0)),
            pl.BlockSpec((ROWS_BLK, D), lambda i: (i, 0)),
            pl.BlockSpec((D, D), lambda i: (0, 0)),
            pl.BlockSpec((1, D), lambda i: (0, 0)),
            pl.BlockSpec((D, D), lambda i: (0, 0)),
            pl.BlockSpec((1, D), lambda i: (0, 0)),
        ],
        out_specs=pl.BlockSpec((ROWS_BLK, D), lambda i: (i, 0)),
        out_shape=jax.ShapeDtypeStruct((N, D), jnp.float32),
    )(p, p, x, wa, ba, wb, bb)


def kernel(x, edge_index, W1a, b1a, W1b, b1b, W2a, b2a, W2b, b2b):
    src = edge_index[0].astype(jnp.int32).reshape(NW, NGRP, GSZ, CHUNK)
    dst = edge_index[1].astype(jnp.int32).reshape(NW, NGRP, GSZ, CHUNK)
    b1a2, b1b2 = b1a.reshape(1, D), b1b.reshape(1, D)
    b2a2, b2b2 = b2a.reshape(1, D), b2b.reshape(1, D)

    p = _sc_aggregate(x, src, dst)
    h1 = _mlp("relu", p, x, W1a, b1a2, W1b, b1b2)
    p2 = _sc_aggregate(h1, src, dst)
    return _mlp("logsoftmax", p2, h1, W2a, b2a2, W2b, b2b2)
